# baseline (speedup 1.0000x reference)
import functools

import jax
import jax.numpy as jnp
from jax.experimental import pallas as pl
from jax.experimental.pallas import tpu as pltpu
from jax.experimental.pallas import tpu_sc as plsc

_N = 512
_HID = 128
_E = _N * _N
_TI = 16
_NI = _N // _TI
_R = _TI * _N
_GW = 256


def _dist2_sc(px, py, pz, idx_flat):
    mesh = plsc.VectorSubcoreMesh(core_axis_name="c", subcore_axis_name="s")

    @pl.kernel(
        out_type=jax.ShapeDtypeStruct((_E // 128, 128), jnp.float32),
        mesh=mesh,
        scratch_types=[pltpu.VMEM((_N,), jnp.float32)] * 3,
        compiler_params=pltpu.CompilerParams(needs_layout_passes=False),
    )
    def k(px_hbm, py_hbm, pz_hbm, i_hbm, o_hbm, sx, sy, sz):
        pltpu.sync_copy(px_hbm, sx)
        pltpu.sync_copy(py_hbm, sy)
        pltpu.sync_copy(pz_hbm, sz)

        def body(i0_vmem, i1_vmem, o_vmem):
            for j in range(_GW // 16):
                s = pl.ds(16 * j, 16)
                a = i0_vmem[0, s]
                b = i1_vmem[0, s]
                dx = plsc.load_gather(sx, [a]) - plsc.load_gather(sx, [b])
                dy = plsc.load_gather(sy, [a]) - plsc.load_gather(sy, [b])
                dz = plsc.load_gather(sz, [a]) - plsc.load_gather(sz, [b])
                o_vmem[16 * j // 128, pl.ds(16 * j % 128, 16)] = (
                    dx * dx + dy * dy + dz * dz
                )

        pltpu.emit_pipeline(
            body,
            grid=(_E // _GW,),
            in_specs=[
                pl.BlockSpec((1, _GW), index_map=lambda i: (0, i)),
                pl.BlockSpec((1, _GW), index_map=lambda i: (0, i + _E // _GW)),
            ],
            out_specs=[
                pl.BlockSpec((_GW // 128, 128), index_map=lambda i: (i, 0))
            ],
            core_axis_name=("c", "s"),
            dimension_semantics=(pltpu.PARALLEL,),
        )(i_hbm, i_hbm, o_hbm)

    return k(px, py, pz, idx_flat)


def _mlp_kernel(x_ref, w0_ref, b0_ref, w1_ref, b1_ref, o_ref):
    t = jnp.maximum(
        jnp.dot(x_ref[...], w0_ref[...], preferred_element_type=jnp.float32)
        + b0_ref[...],
        0.0,
    )
    o_ref[...] = (
        jnp.dot(t, w1_ref[...], preferred_element_type=jnp.float32) + b1_ref[...]
    )


def _mlp_call(x, w0t, b0, w1t, b1, out_rows, out_cols):
    return pl.pallas_call(
        _mlp_kernel,
        out_shape=jax.ShapeDtypeStruct((out_rows, out_cols), jnp.float32),
    )(x, w0t, b0, w1t, b1)


def _edge_kernel(
    h_ref, pos_ref, d_ref, rsel_ref, lmask_ref,
    wa_ref, wb_ref, wd2_ref, b0_ref, w1_ref, b1_ref,
    ew_ref, eb_ref, pw0_ref, pb0_ref, pw1_ref, pb1_ref,
    om_ref, osp_ref, os0_ref,
):
    i = pl.program_id(0)
    h = h_ref[...].astype(jnp.bfloat16)
    a = jnp.dot(h, wa_ref[...], preferred_element_type=jnp.float32) + b0_ref[...]
    hi = h_ref[pl.ds(i * _TI, _TI), :].astype(jnp.bfloat16)
    b = jnp.dot(hi, wb_ref[...], preferred_element_type=jnp.float32)

    ds_ = jnp.sqrt(d_ref[...]).astype(jnp.bfloat16)
    t_rows = jnp.dot(
        rsel_ref[...], ds_, preferred_element_type=jnp.float32
    )
    dterm = jnp.dot(
        (t_rows * lmask_ref[...]).astype(jnp.bfloat16), wd2_ref[...],
        preferred_element_type=jnp.float32,
    )

    pre = (b[:, None, :] + a[None, :, :]).reshape(_R, _HID) + dterm
    t = jnp.maximum(pre, 0.0).astype(jnp.bfloat16)
    m1 = jnp.dot(t, w1_ref[...], preferred_element_type=jnp.float32) + b1_ref[...]
    g = jax.nn.sigmoid(
        jnp.dot(
            m1.astype(jnp.bfloat16), ew_ref[...],
            preferred_element_type=jnp.float32,
        )
        + eb_ref[0:1, 0:1]
    )
    m = m1 * g
    u = jnp.maximum(
        jnp.dot(
            m.astype(jnp.bfloat16), pw0_ref[...],
            preferred_element_type=jnp.float32,
        )
        + pb0_ref[...],
        0.0,
    )
    s = (
        jnp.dot(u, pw1_ref[...], preferred_element_type=jnp.float32)
        + pb1_ref[0:1, 0:1]
    )

    m_sum = jnp.sum(m.reshape(_TI, _N, _HID), axis=0)
    s3 = s.reshape(_TI, _N, 1)
    s0_sum = jnp.sum(s3, axis=0)
    posi = pos_ref[pl.ds(i * _TI, _TI), :]
    sp_sum = jnp.sum(s3 * posi[:, None, :], axis=0)

    @pl.when(i == 0)
    def _():
        om_ref[...] = m_sum
        osp_ref[...] = sp_sum
        os0_ref[...] = s0_sum

    @pl.when(i > 0)
    def _():
        om_ref[...] += m_sum
        osp_ref[...] += sp_sum
        os0_ref[...] += s0_sum


def _edge_call(h, pos4, dist, rsel, lmask, lw):
    full = lambda shape: pl.BlockSpec(shape, lambda i: (0, 0))
    return pl.pallas_call(
        _edge_kernel,
        grid=(_NI,),
        in_specs=[
            full((_N, _HID)),
            full((_N, 4)),
            pl.BlockSpec((_R // 128, 128), lambda i: (i, 0)),
            full((_R, _R // 128)),
            full((_R, 128)),
            full((_HID, _HID)), full((_HID, _HID)), full((_HID, _HID)),
            full((1, _HID)), full((_HID, _HID)), full((1, _HID)),
            full((_HID, 1)), full((1, 1)),
            full((_HID, _HID)), full((1, _HID)), full((_HID, 1)), full((1, 1)),
        ],
        out_specs=[
            pl.BlockSpec((_N, _HID), lambda i: (0, 0)),
            pl.BlockSpec((_N, 4), lambda i: (0, 0)),
            pl.BlockSpec((_N, 1), lambda i: (0, 0)),
        ],
        out_shape=[
            jax.ShapeDtypeStruct((_N, _HID), jnp.float32),
            jax.ShapeDtypeStruct((_N, 4), jnp.float32),
            jax.ShapeDtypeStruct((_N, 1), jnp.float32),
        ],
        compiler_params=pltpu.CompilerParams(
            dimension_semantics=("arbitrary",)
        ),
    )(
        h, pos4, dist, rsel, lmask,
        lw["wa"], lw["wb"], lw["wd2"], lw["b0"], lw["w1t"], lw["b1"],
        lw["ew"], lw["eb"], lw["pw0t"], lw["pb0"], lw["pw1t"], lw["pb1"],
    )


def _node_kernel(
    h_ref, om_ref, osp_ref, os0_ref, pos_ref,
    u1_ref, u2_ref, ub0_ref, uw1_ref, ub1_ref,
    ho_ref, po_ref,
):
    inv_n = 1.0 / _N
    h = h_ref[...]
    nm = om_ref[...] * inv_n
    t = jnp.maximum(
        jnp.dot(h, u1_ref[...], preferred_element_type=jnp.float32)
        + jnp.dot(nm, u2_ref[...], preferred_element_type=jnp.float32)
        + ub0_ref[...],
        0.0,
    )
    ho_ref[...] = (
        jnp.dot(t, uw1_ref[...], preferred_element_type=jnp.float32) + ub1_ref[...]
    )
    pos = pos_ref[...]
    po_ref[...] = pos + (pos * os0_ref[...] - osp_ref[...]) * inv_n


def _node_call(h, om, osp, os0, pos4, lw):
    return pl.pallas_call(
        _node_kernel,
        out_shape=[
            jax.ShapeDtypeStruct((_N, _HID), jnp.float32),
            jax.ShapeDtypeStruct((_N, 4), jnp.float32),
        ],
    )(h, om, osp, os0, pos4, lw["u1"], lw["u2"], lw["ub0"], lw["uw1t"], lw["ub1"])


def _pool_kernel(
    h_ref, b_ref, w0_ref, b0_ref, w1_ref, b1_ref, o_ref, *, num_graphs
):
    gi = jax.lax.broadcasted_iota(jnp.int32, (num_graphs, _N), 0)
    mask = (b_ref[...] == gi).astype(jnp.float32)
    pooled = jnp.dot(mask, h_ref[...], preferred_element_type=jnp.float32)
    t = jnp.maximum(
        jnp.dot(pooled, w0_ref[...], preferred_element_type=jnp.float32)
        + b0_ref[...],
        0.0,
    )
    o_ref[...] = (
        jnp.dot(t, w1_ref[...], preferred_element_type=jnp.float32) + b1_ref[...]
    )


def _pool_call(h, batch2d, hw0t, hb0, hw1t, hb1, num_graphs, out_f):
    return pl.pallas_call(
        functools.partial(_pool_kernel, num_graphs=num_graphs),
        out_shape=jax.ShapeDtypeStruct((num_graphs, out_f), jnp.float32),
    )(h, batch2d, hw0t, hb0, hw1t, hb1)


def _prep_layer(lp):
    w0 = lp["msg_w0"]
    bf = jnp.bfloat16
    return {
        "wa": jnp.transpose(w0[:, :_HID]).astype(bf),
        "wb": jnp.transpose(w0[:, _HID : 2 * _HID]).astype(bf),
        "wd2": jnp.broadcast_to(w0[:, 2 * _HID][None, :], (128, _HID)).astype(bf),
        "b0": lp["msg_b0"][None, :],
        "w1t": jnp.transpose(lp["msg_w1"]).astype(bf),
        "b1": lp["msg_b1"][None, :],
        "ew": jnp.transpose(lp["edge_w"]).astype(bf),
        "eb": lp["edge_b"][None, :],
        "pw0t": jnp.transpose(lp["pos_w0"]).astype(bf),
        "pb0": lp["pos_b0"][None, :],
        "pw1t": jnp.transpose(lp["pos_w1"]),
        "pb1": lp["pos_b1"][None, :],
        "u1": jnp.transpose(lp["upd_w0"][:, :_HID]),
        "u2": jnp.transpose(lp["upd_w0"][:, _HID:]),
        "ub0": lp["upd_b0"][None, :],
        "uw1t": jnp.transpose(lp["upd_w1"]),
        "ub1": lp["upd_b1"][None, :],
    }


def kernel(x, pos, edge_index, batch, params):
    x = x.astype(jnp.float32)
    pos = pos.astype(jnp.float32)
    n, node_f = x.shape

    idx_flat = edge_index.astype(jnp.int32).reshape(1, 2 * _E)
    dist = _dist2_sc(pos[:, 0], pos[:, 1], pos[:, 2], idx_flat)

    e = params["emb"]
    h = _mlp_call(
        x,
        jnp.transpose(e["w0"]), e["b0"][None, :],
        jnp.transpose(e["w1"]), e["b1"][None, :],
        n, _HID,
    )

    pos4 = jnp.pad(pos, ((0, 0), (0, 1)))

    r_iota = jnp.arange(_R, dtype=jnp.int32)
    rsel = (
        (r_iota[:, None] // 128) == jnp.arange(_R // 128, dtype=jnp.int32)
    ).astype(jnp.bfloat16)
    lmask = (
        (r_iota[:, None] % 128) == jnp.arange(128, dtype=jnp.int32)
    ).astype(jnp.bfloat16)

    for lp in params["layers"]:
        lw = _prep_layer(lp)
        om, osp, os0 = _edge_call(h, pos4, dist, rsel, lmask, lw)
        h, pos4 = _node_call(h, om, osp, os0, pos4, lw)

    hd = params["head"]
    num_graphs = 16
    out_f = hd["w1"].shape[0]
    batch2d = batch.astype(jnp.int32).reshape(1, n)
    return _pool_call(
        h,
        batch2d,
        jnp.transpose(hd["w0"]), hd["b0"][None, :],
        jnp.transpose(hd["w1"]), hd["b1"][None, :],
        num_graphs, out_f,
    )


# speedup vs baseline: 1.0786x; 1.0786x over previous
import functools

import jax
import jax.numpy as jnp
from jax.experimental import pallas as pl
from jax.experimental.pallas import tpu as pltpu
from jax.experimental.pallas import tpu_sc as plsc

_N = 512
_HID = 128
_E = _N * _N
_TI = 16
_NI = _N // _TI
_R = _TI * _N
_GW = 256


def _dist2_sc(px, py, pz, idx_flat):
    mesh = plsc.VectorSubcoreMesh(core_axis_name="c", subcore_axis_name="s")

    @pl.kernel(
        out_type=jax.ShapeDtypeStruct((_E // 128, 128), jnp.float32),
        mesh=mesh,
        scratch_types=[pltpu.VMEM((_N,), jnp.float32)] * 3,
        compiler_params=pltpu.CompilerParams(needs_layout_passes=False),
    )
    def k(px_hbm, py_hbm, pz_hbm, i_hbm, o_hbm, sx, sy, sz):
        pltpu.sync_copy(px_hbm, sx)
        pltpu.sync_copy(py_hbm, sy)
        pltpu.sync_copy(pz_hbm, sz)

        def body(i0_vmem, i1_vmem, o_vmem):
            for j in range(_GW // 16):
                s = pl.ds(16 * j, 16)
                a = i0_vmem[0, s]
                b = i1_vmem[0, s]
                dx = plsc.load_gather(sx, [a]) - plsc.load_gather(sx, [b])
                dy = plsc.load_gather(sy, [a]) - plsc.load_gather(sy, [b])
                dz = plsc.load_gather(sz, [a]) - plsc.load_gather(sz, [b])
                o_vmem[16 * j // 128, pl.ds(16 * j % 128, 16)] = (
                    dx * dx + dy * dy + dz * dz
                )

        pltpu.emit_pipeline(
            body,
            grid=(_E // _GW,),
            in_specs=[
                pl.BlockSpec((1, _GW), index_map=lambda i: (0, i)),
                pl.BlockSpec((1, _GW), index_map=lambda i: (0, i + _E // _GW)),
            ],
            out_specs=[
                pl.BlockSpec((_GW // 128, 128), index_map=lambda i: (i, 0))
            ],
            core_axis_name=("c", "s"),
            dimension_semantics=(pltpu.PARALLEL,),
        )(i_hbm, i_hbm, o_hbm)

    return k(px, py, pz, idx_flat)


def _mlp_kernel(x_ref, w0_ref, b0_ref, w1_ref, b1_ref, o_ref):
    t = jnp.maximum(
        jnp.dot(x_ref[...], w0_ref[...], preferred_element_type=jnp.float32)
        + b0_ref[...],
        0.0,
    )
    o_ref[...] = (
        jnp.dot(t, w1_ref[...], preferred_element_type=jnp.float32) + b1_ref[...]
    )


def _mlp_call(x, w0t, b0, w1t, b1, out_rows, out_cols):
    return pl.pallas_call(
        _mlp_kernel,
        out_shape=jax.ShapeDtypeStruct((out_rows, out_cols), jnp.float32),
    )(x, w0t, b0, w1t, b1)


def _edge_kernel(
    h_ref, pos_ref, d_ref, rsel_ref, lmask_ref,
    wa_ref, wb_ref, wd2_ref, b0_ref, w1_ref, b1_ref,
    ew_ref, eb_ref, pw0_ref, pb0_ref, pw1_ref, pb1_ref,
    om_ref, osp_ref,
):
    i = pl.program_id(0)
    h = h_ref[...].astype(jnp.bfloat16)
    a = jnp.dot(h, wa_ref[...], preferred_element_type=jnp.float32) + b0_ref[...]
    hi = h_ref[pl.ds(i * _TI, _TI), :].astype(jnp.bfloat16)
    b = jnp.dot(hi, wb_ref[...], preferred_element_type=jnp.float32)

    ds_ = jnp.sqrt(d_ref[...]).astype(jnp.bfloat16)
    t_rows = jnp.dot(
        rsel_ref[...], ds_, preferred_element_type=jnp.float32
    ).astype(jnp.bfloat16)
    dterm = jnp.dot(
        t_rows * lmask_ref[...], wd2_ref[...],
        preferred_element_type=jnp.float32,
    )

    pre = (b[:, None, :] + a[None, :, :]).reshape(_R, _HID) + dterm
    t = jnp.maximum(pre, 0.0).astype(jnp.bfloat16)
    m1 = jnp.dot(t, w1_ref[...], preferred_element_type=jnp.float32) + b1_ref[...]
    g = jax.nn.sigmoid(
        jnp.dot(
            m1.astype(jnp.bfloat16), ew_ref[...],
            preferred_element_type=jnp.float32,
        )
        + eb_ref[0:1, 0:1]
    )
    m = m1 * g
    u = jnp.maximum(
        jnp.dot(
            m.astype(jnp.bfloat16), pw0_ref[...],
            preferred_element_type=jnp.float32,
        )
        + pb0_ref[...],
        0.0,
    )
    s_full = (
        jnp.dot(
            u.astype(jnp.bfloat16), pw1_ref[...],
            preferred_element_type=jnp.float32,
        )
        + pb1_ref[0:1, 0:1]
    )

    m_sum = jnp.sum(m.reshape(_TI, _N, _HID), axis=0)
    s3 = s_full.reshape(_TI, _N, _HID)[:, :, :4]
    posi = pos_ref[pl.ds(i * _TI, _TI), :]
    sp_sum = jnp.sum(s3 * posi[:, None, :], axis=0)

    @pl.when(i == 0)
    def _():
        om_ref[...] = m_sum
        osp_ref[...] = sp_sum

    @pl.when(i > 0)
    def _():
        om_ref[...] += m_sum
        osp_ref[...] += sp_sum


def _edge_call(h, pos4, dist, rsel, lmask, lw):
    full = lambda shape: pl.BlockSpec(shape, lambda i: (0, 0))
    return pl.pallas_call(
        _edge_kernel,
        grid=(_NI,),
        in_specs=[
            full((_N, _HID)),
            full((_N, 4)),
            pl.BlockSpec((_R // 128, 128), lambda i: (i, 0)),
            full((_R, _R // 128)),
            full((_R, 128)),
            full((_HID, _HID)), full((_HID, _HID)), full((_HID, _HID)),
            full((1, _HID)), full((_HID, _HID)), full((1, _HID)),
            full((_HID, _HID)), full((1, 1)),
            full((_HID, _HID)), full((1, _HID)), full((_HID, _HID)), full((1, 1)),
        ],
        out_specs=[
            pl.BlockSpec((_N, _HID), lambda i: (0, 0)),
            pl.BlockSpec((_N, 4), lambda i: (0, 0)),
        ],
        out_shape=[
            jax.ShapeDtypeStruct((_N, _HID), jnp.float32),
            jax.ShapeDtypeStruct((_N, 4), jnp.float32),
        ],
        compiler_params=pltpu.CompilerParams(
            dimension_semantics=("arbitrary",)
        ),
    )(
        h, pos4, dist, rsel, lmask,
        lw["wa"], lw["wb"], lw["wd2"], lw["b0"], lw["w1t"], lw["b1"],
        lw["ew2"], lw["eb"], lw["pw0t"], lw["pb0"], lw["pw12"], lw["pb1"],
    )


def _node_kernel(
    h_ref, om_ref, osp_ref, pos_ref,
    u1_ref, u2_ref, ub0_ref, uw1_ref, ub1_ref,
    ho_ref, po_ref,
):
    inv_n = 1.0 / _N
    h = h_ref[...]
    nm = om_ref[...] * inv_n
    t = jnp.maximum(
        jnp.dot(h, u1_ref[...], preferred_element_type=jnp.float32)
        + jnp.dot(nm, u2_ref[...], preferred_element_type=jnp.float32)
        + ub0_ref[...],
        0.0,
    )
    ho_ref[...] = (
        jnp.dot(t, uw1_ref[...], preferred_element_type=jnp.float32) + ub1_ref[...]
    )
    pos = pos_ref[...]
    osp = osp_ref[...]
    po_ref[...] = pos + (pos * osp[:, 3:4] - osp) * inv_n


def _node_call(h, om, osp, pos4, lw):
    return pl.pallas_call(
        _node_kernel,
        out_shape=[
            jax.ShapeDtypeStruct((_N, _HID), jnp.float32),
            jax.ShapeDtypeStruct((_N, 4), jnp.float32),
        ],
    )(h, om, osp, pos4, lw["u1"], lw["u2"], lw["ub0"], lw["uw1t"], lw["ub1"])


def _pool_kernel(
    h_ref, b_ref, w0_ref, b0_ref, w1_ref, b1_ref, o_ref, *, num_graphs
):
    gi = jax.lax.broadcasted_iota(jnp.int32, (num_graphs, _N), 0)
    mask = (b_ref[...] == gi).astype(jnp.float32)
    pooled = jnp.dot(mask, h_ref[...], preferred_element_type=jnp.float32)
    t = jnp.maximum(
        jnp.dot(pooled, w0_ref[...], preferred_element_type=jnp.float32)
        + b0_ref[...],
        0.0,
    )
    o_ref[...] = (
        jnp.dot(t, w1_ref[...], preferred_element_type=jnp.float32) + b1_ref[...]
    )


def _pool_call(h, batch2d, hw0t, hb0, hw1t, hb1, num_graphs, out_f):
    return pl.pallas_call(
        functools.partial(_pool_kernel, num_graphs=num_graphs),
        out_shape=jax.ShapeDtypeStruct((num_graphs, out_f), jnp.float32),
    )(h, batch2d, hw0t, hb0, hw1t, hb1)


def _prep_layer(lp):
    w0 = lp["msg_w0"]
    bf = jnp.bfloat16
    return {
        "wa": jnp.transpose(w0[:, :_HID]).astype(bf),
        "wb": jnp.transpose(w0[:, _HID : 2 * _HID]).astype(bf),
        "wd2": jnp.broadcast_to(w0[:, 2 * _HID][None, :], (128, _HID)).astype(bf),
        "b0": lp["msg_b0"][None, :],
        "w1t": jnp.transpose(lp["msg_w1"]).astype(bf),
        "b1": lp["msg_b1"][None, :],
        "ew2": jnp.broadcast_to(
            jnp.transpose(lp["edge_w"]), (_HID, 128)
        ).astype(bf),
        "eb": lp["edge_b"][None, :],
        "pw0t": jnp.transpose(lp["pos_w0"]).astype(bf),
        "pb0": lp["pos_b0"][None, :],
        "pw12": jnp.broadcast_to(
            jnp.transpose(lp["pos_w1"]), (_HID, 128)
        ).astype(bf),
        "pb1": lp["pos_b1"][None, :],
        "u1": jnp.transpose(lp["upd_w0"][:, :_HID]),
        "u2": jnp.transpose(lp["upd_w0"][:, _HID:]),
        "ub0": lp["upd_b0"][None, :],
        "uw1t": jnp.transpose(lp["upd_w1"]),
        "ub1": lp["upd_b1"][None, :],
    }


def kernel(x, pos, edge_index, batch, params):
    x = x.astype(jnp.float32)
    pos = pos.astype(jnp.float32)
    n, node_f = x.shape

    idx_flat = edge_index.astype(jnp.int32).reshape(1, 2 * _E)
    dist = _dist2_sc(pos[:, 0], pos[:, 1], pos[:, 2], idx_flat)

    e = params["emb"]
    h = _mlp_call(
        x,
        jnp.transpose(e["w0"]), e["b0"][None, :],
        jnp.transpose(e["w1"]), e["b1"][None, :],
        n, _HID,
    )

    pos4 = jnp.concatenate(
        [pos, jnp.ones((n, 1), jnp.float32)], axis=1
    )

    r_iota = jnp.arange(_R, dtype=jnp.int32)
    rsel = (
        (r_iota[:, None] // 128) == jnp.arange(_R // 128, dtype=jnp.int32)
    ).astype(jnp.bfloat16)
    lmask = (
        (r_iota[:, None] % 128) == jnp.arange(128, dtype=jnp.int32)
    ).astype(jnp.bfloat16)

    for lp in params["layers"]:
        lw = _prep_layer(lp)
        om, osp = _edge_call(h, pos4, dist, rsel, lmask, lw)
        h, pos4 = _node_call(h, om, osp, pos4, lw)

    hd = params["head"]
    num_graphs = 16
    out_f = hd["w1"].shape[0]
    batch2d = batch.astype(jnp.int32).reshape(1, n)
    return _pool_call(
        h,
        batch2d,
        jnp.transpose(hd["w0"]), hd["b0"][None, :],
        jnp.transpose(hd["w1"]), hd["b1"][None, :],
        num_graphs, out_f,
    )


# speedup vs baseline: 1.1252x; 1.0432x over previous
import functools

import jax
import jax.numpy as jnp
from jax.experimental import pallas as pl
from jax.experimental.pallas import tpu as pltpu
from jax.experimental.pallas import tpu_sc as plsc

_N = 512
_HID = 128
_E = _N * _N
_TI = 16
_NI = _N // _TI
_R = _TI * _N
_GW = 256


def _dist2_sc(px, py, pz, idx_flat):
    mesh = plsc.VectorSubcoreMesh(core_axis_name="c", subcore_axis_name="s")

    @pl.kernel(
        out_type=jax.ShapeDtypeStruct((_NI, _R), jnp.float32),
        mesh=mesh,
        scratch_types=[pltpu.VMEM((_N,), jnp.float32)] * 3,
        compiler_params=pltpu.CompilerParams(needs_layout_passes=False),
    )
    def k(px_hbm, py_hbm, pz_hbm, i_hbm, o_hbm, sx, sy, sz):
        pltpu.sync_copy(px_hbm, sx)
        pltpu.sync_copy(py_hbm, sy)
        pltpu.sync_copy(pz_hbm, sz)

        def body(i0_vmem, i1_vmem, o_vmem):
            for j in range(_GW // 16):
                s = pl.ds(16 * j, 16)
                a = i0_vmem[0, s]
                b = i1_vmem[0, s]
                dx = plsc.load_gather(sx, [a]) - plsc.load_gather(sx, [b])
                dy = plsc.load_gather(sy, [a]) - plsc.load_gather(sy, [b])
                dz = plsc.load_gather(sz, [a]) - plsc.load_gather(sz, [b])
                o_vmem[0, s] = dx * dx + dy * dy + dz * dz

        blocks_per_row = _R // _GW
        pltpu.emit_pipeline(
            body,
            grid=(_E // _GW,),
            in_specs=[
                pl.BlockSpec((1, _GW), index_map=lambda i: (0, i)),
                pl.BlockSpec((1, _GW), index_map=lambda i: (0, i + _E // _GW)),
            ],
            out_specs=[
                pl.BlockSpec(
                    (1, _GW),
                    index_map=lambda i: (
                        i // blocks_per_row,
                        i % blocks_per_row,
                    ),
                )
            ],
            core_axis_name=("c", "s"),
            dimension_semantics=(pltpu.PARALLEL,),
        )(i_hbm, i_hbm, o_hbm)

    return k(px, py, pz, idx_flat)


def _mlp_t_kernel(x_ref, w0_ref, b0_ref, w1_ref, b1_ref, o_ref):
    t = jnp.maximum(
        jnp.dot(w0_ref[...], x_ref[...], preferred_element_type=jnp.float32)
        + b0_ref[...],
        0.0,
    )
    o_ref[...] = (
        jnp.dot(w1_ref[...], t, preferred_element_type=jnp.float32) + b1_ref[...]
    )


def _mlp_t_call(x, w0, b0c, w1, b1c, out_rows, out_cols):
    return pl.pallas_call(
        _mlp_t_kernel,
        out_shape=jax.ShapeDtypeStruct((out_rows, out_cols), jnp.float32),
    )(x, w0, b0c, w1, b1c)


def _edge_kernel(
    h_ref, pos_ref, d_ref, rep_ref,
    wa_ref, wb_ref, wdc_ref, b0_ref, w1_ref, b1_ref,
    wg_ref, cg_ref, pw0_ref, pb0_ref, pw1_ref, pb1_ref,
    om_ref, osp_ref,
):
    i = pl.program_id(0)
    hb = h_ref[...].astype(jnp.bfloat16)
    a = (
        jnp.dot(wa_ref[...], hb, preferred_element_type=jnp.float32)
        + b0_ref[...]
    )
    sel = (
        jax.lax.broadcasted_iota(jnp.int32, (_N, _TI), 0)
        == _TI * i + jax.lax.broadcasted_iota(jnp.int32, (_N, _TI), 1)
    ).astype(jnp.bfloat16)
    hi = jnp.dot(hb, sel, preferred_element_type=jnp.float32).astype(
        jnp.bfloat16
    )
    b = jnp.dot(
        wb_ref[...], hi, preferred_element_type=jnp.float32
    ).astype(jnp.bfloat16)
    posb = jnp.dot(
        pos_ref[...].astype(jnp.bfloat16), sel,
        preferred_element_type=jnp.float32,
    )

    dline = jnp.sqrt(d_ref[pl.ds(i, 1), :])
    a16 = jnp.concatenate([a] * _TI, axis=1)
    bfull = jnp.dot(
        b, rep_ref[...], preferred_element_type=jnp.float32
    )
    pre = a16 + bfull + wdc_ref[...] * dline
    t = jnp.maximum(pre, 0.0).astype(jnp.bfloat16)
    m1 = (
        jnp.dot(w1_ref[...], t, preferred_element_type=jnp.float32)
        + b1_ref[...]
    )
    glog = (
        jnp.dot(wg_ref[...], t, preferred_element_type=jnp.float32)
        + cg_ref[0:1, 0:1]
    )
    m = m1 * jax.nn.sigmoid(glog)
    u = jnp.maximum(
        jnp.dot(
            pw0_ref[...], m.astype(jnp.bfloat16),
            preferred_element_type=jnp.float32,
        )
        + pb0_ref[...],
        0.0,
    )
    s = (
        jnp.dot(
            pw1_ref[...], u.astype(jnp.bfloat16),
            preferred_element_type=jnp.float32,
        )
        + pb1_ref[0:1, 0:1]
    )

    m_sum = jnp.zeros((_HID, _N), jnp.float32)
    sp_sum = jnp.zeros((4, _N), jnp.float32)
    for ti in range(_TI):
        cols = slice(ti * _N, (ti + 1) * _N)
        m_sum = m_sum + m[:, cols]
        sp_sum = sp_sum + posb[:, ti : ti + 1] * s[:, cols]

    @pl.when(i == 0)
    def _():
        om_ref[...] = m_sum
        osp_ref[...] = sp_sum

    @pl.when(i > 0)
    def _():
        om_ref[...] += m_sum
        osp_ref[...] += sp_sum


def _edge_call(h, pos4, dist, rep, lw):
    full = lambda shape: pl.BlockSpec(shape, lambda i: (0, 0))
    return pl.pallas_call(
        _edge_kernel,
        grid=(_NI,),
        in_specs=[
            full((_HID, _N)),
            full((4, _N)),
            full((_NI, _R)),
            full((_TI, _R)),
            full((_HID, _HID)), full((_HID, _HID)), full((_HID, 1)),
            full((_HID, 1)), full((_HID, _HID)), full((_HID, 1)),
            full((1, _HID)), full((1, 1)),
            full((_HID, _HID)), full((_HID, 1)), full((1, _HID)), full((1, 1)),
        ],
        out_specs=[
            pl.BlockSpec((_HID, _N), lambda i: (0, 0)),
            pl.BlockSpec((4, _N), lambda i: (0, 0)),
        ],
        out_shape=[
            jax.ShapeDtypeStruct((_HID, _N), jnp.float32),
            jax.ShapeDtypeStruct((4, _N), jnp.float32),
        ],
        compiler_params=pltpu.CompilerParams(
            dimension_semantics=("arbitrary",)
        ),
    )(
        h, pos4, dist, rep,
        lw["wa"], lw["wb"], lw["wdc"], lw["b0c"], lw["w1"], lw["b1c"],
        lw["wg"], lw["cg"], lw["pw0"], lw["pb0c"], lw["pw1"], lw["pb1"],
    )


def _node_kernel(
    h_ref, om_ref, osp_ref, pos_ref,
    u1_ref, u2_ref, ub0_ref, uw1_ref, ub1_ref,
    ho_ref, po_ref,
):
    inv_n = 1.0 / _N
    h = h_ref[...]
    nm = om_ref[...] * inv_n
    t = jnp.maximum(
        jnp.dot(u1_ref[...], h, preferred_element_type=jnp.float32)
        + jnp.dot(u2_ref[...], nm, preferred_element_type=jnp.float32)
        + ub0_ref[...],
        0.0,
    )
    ho_ref[...] = (
        jnp.dot(uw1_ref[...], t, preferred_element_type=jnp.float32)
        + ub1_ref[...]
    )
    pos = pos_ref[...]
    osp = osp_ref[...]
    po_ref[...] = pos + (pos * osp[3:4, :] - osp) * inv_n


def _node_call(h, om, osp, pos4, lw):
    return pl.pallas_call(
        _node_kernel,
        out_shape=[
            jax.ShapeDtypeStruct((_HID, _N), jnp.float32),
            jax.ShapeDtypeStruct((4, _N), jnp.float32),
        ],
    )(h, om, osp, pos4, lw["u1"], lw["u2"], lw["ub0c"], lw["uw1"], lw["ub1c"])


def _pool_kernel(
    h_ref, b_ref, w0_ref, b0_ref, w1_ref, b1_ref, o_ref, *, num_graphs
):
    gi = jax.lax.broadcasted_iota(jnp.int32, (_N, num_graphs), 1)
    mask = (b_ref[...] == gi).astype(jnp.float32)
    pooled = jnp.dot(
        h_ref[...], mask, preferred_element_type=jnp.float32
    )
    t = jnp.maximum(
        jnp.dot(w0_ref[...], pooled, preferred_element_type=jnp.float32)
        + b0_ref[...],
        0.0,
    )
    o_ref[...] = (
        jnp.dot(w1_ref[...], t, preferred_element_type=jnp.float32)
        + b1_ref[...]
    )


def _pool_call(h, batchcol, w0, b0c, w1, b1c, num_graphs, out_f):
    return pl.pallas_call(
        functools.partial(_pool_kernel, num_graphs=num_graphs),
        out_shape=jax.ShapeDtypeStruct((out_f, num_graphs), jnp.float32),
    )(h, batchcol, w0, b0c, w1, b1c)


def _prep_layer(lp):
    w0 = lp["msg_w0"]
    bf = jnp.bfloat16
    return {
        "wa": w0[:, :_HID].astype(bf),
        "wb": w0[:, _HID : 2 * _HID].astype(bf),
        "wdc": w0[:, 2 * _HID : 2 * _HID + 1],
        "b0c": lp["msg_b0"][:, None],
        "w1": lp["msg_w1"].astype(bf),
        "b1c": lp["msg_b1"][:, None],
        "wg": (lp["edge_w"] @ lp["msg_w1"]).astype(bf),
        "cg": (lp["edge_w"] @ lp["msg_b1"][:, None]) + lp["edge_b"][None, :],
        "pw0": lp["pos_w0"].astype(bf),
        "pb0c": lp["pos_b0"][:, None],
        "pw1": lp["pos_w1"].astype(bf),
        "pb1": lp["pos_b1"][None, :],
        "u1": lp["upd_w0"][:, :_HID],
        "u2": lp["upd_w0"][:, _HID:],
        "ub0c": lp["upd_b0"][:, None],
        "uw1": lp["upd_w1"],
        "ub1c": lp["upd_b1"][:, None],
    }


def kernel(x, pos, edge_index, batch, params):
    x = x.astype(jnp.float32)
    pos = pos.astype(jnp.float32)
    n = x.shape[0]

    idx_flat = edge_index.astype(jnp.int32).reshape(1, 2 * _E)
    dist = _dist2_sc(pos[:, 0], pos[:, 1], pos[:, 2], idx_flat)

    e = params["emb"]
    h = _mlp_t_call(
        jnp.transpose(x),
        e["w0"], e["b0"][:, None],
        e["w1"], e["b1"][:, None],
        _HID, n,
    )

    pos4 = jnp.concatenate(
        [jnp.transpose(pos), jnp.ones((1, n), jnp.float32)], axis=0
    )

    rep = (
        jnp.arange(_TI, dtype=jnp.int32)[:, None]
        == (jnp.arange(_R, dtype=jnp.int32) // _N)
    ).astype(jnp.bfloat16)

    for lp in params["layers"]:
        lw = _prep_layer(lp)
        om, osp = _edge_call(h, pos4, dist, rep, lw)
        h, pos4 = _node_call(h, om, osp, pos4, lw)

    hd = params["head"]
    num_graphs = 16
    out_f = hd["w1"].shape[0]
    batchcol = batch.astype(jnp.int32).reshape(n, 1)
    out_t = _pool_call(
        h, batchcol,
        hd["w0"], hd["b0"][:, None],
        hd["w1"], hd["b1"][:, None],
        num_graphs, out_f,
    )
    return jnp.transpose(out_t)
